# baseline (speedup 1.0000x reference)
import jax
import jax.numpy as jnp
from jax.experimental import pallas as pl
from jax.experimental.pallas import tpu as pltpu


def _copy_body(rows_ref, perm_ref, src_ref, out_ref):
    out_ref[...] = src_ref[...]


def kernel(prefix, embedding):
    B, S = prefix.shape
    V, D = embedding.shape
    N = B * S
    idx = prefix.reshape(N).astype(jnp.int32)
    order = jnp.argsort(idx).astype(jnp.int32)
    rows_sorted = jnp.take(idx, order)
    emb3 = embedding.reshape(V, 1, D)

    out = pl.pallas_call(
        _copy_body,
        grid_spec=pltpu.PrefetchScalarGridSpec(
            num_scalar_prefetch=2,
            grid=(N,),
            in_specs=[
                pl.BlockSpec((1, 1, D), lambda i, rows, perm: (rows[i], 0, 0)),
            ],
            out_specs=pl.BlockSpec((1, 1, D), lambda i, rows, perm: (perm[i], 0, 0)),
        ),
        out_shape=jax.ShapeDtypeStruct((N, 1, D), jnp.float32),
    )(rows_sorted, order, emb3)
    return out.reshape(B, S, D)


# speedup vs baseline: 1.3985x; 1.3985x over previous
import functools

import jax
import jax.numpy as jnp
from jax import lax
from jax.experimental import pallas as pl
from jax.experimental.pallas import tpu as pltpu
from jax.experimental.pallas import tpu_sc as plsc

V = 200
D = 98304
NDEST = 1600
DC = 2048
NCH = D // DC
NT = NDEST * NCH
GRP = 16
NG = NT // GRP
NW = 32
GPW = NG // NW
CPG = NCH // GRP


def _make_sc_call():
    mesh = plsc.VectorSubcoreMesh(core_axis_name="c", subcore_axis_name="s")

    @functools.partial(
        pl.kernel,
        mesh=mesh,
        out_type=jax.ShapeDtypeStruct((NT, DC), jnp.float32),
        scratch_types=[
            pltpu.VMEM((GPW * GRP,), jnp.int32),
            pltpu.VMEM((2, GRP, DC), jnp.float32),
            pltpu.SemaphoreType.DMA,
            pltpu.SemaphoreType.DMA,
            pltpu.SemaphoreType.DMA,
            pltpu.SemaphoreType.DMA,
        ],
    )
    def sc_gather(idx_hbm, table_hbm, out_hbm, idx_v, bufs, g0, g1, s0, s1):
        gsems = (g0, g1)
        ssems = (s0, s1)
        w = lax.axis_index("s") * 2 + lax.axis_index("c")
        base = w * GPW
        pltpu.sync_copy(idx_hbm.at[pl.ds(base * GRP, GPW * GRP)], idx_v)

        def gather(k, b):
            g = k - base
            cp = pltpu.make_async_copy(
                table_hbm.at[idx_v.at[pl.ds(g * GRP, GRP)]], bufs.at[b], gsems[b]
            )
            cp.start()
            cp.wait()

        def scatter_start(k, b):
            pltpu.make_async_copy(
                bufs.at[b], out_hbm.at[pl.ds(k * GRP, GRP)], ssems[b]
            ).start()

        def scatter_wait(k, b):
            pltpu.make_async_copy(
                bufs.at[b], out_hbm.at[pl.ds(k * GRP, GRP)], ssems[b]
            ).wait()

        for b in range(2):
            gather(base + b, b)
            scatter_start(base + b, b)

        def body(i, carry):
            for b in range(2):
                k = base + 2 * i + b
                scatter_wait(k - 2, b)
                gather(k, b)
                scatter_start(k, b)
            return carry

        lax.fori_loop(1, GPW // 2, body, 0)

        for b in range(2):
            scatter_wait(base + GPW - 2 + b, b)

    return sc_gather


_SC_GATHER = _make_sc_call()


def kernel(prefix, embedding):
    B, S = prefix.shape
    idx = prefix.reshape(B * S).astype(jnp.int32)
    src = (idx[:, None] * NCH + jnp.arange(NCH, dtype=jnp.int32)[None, :]).reshape(NT)
    table = embedding.reshape(V * NCH, DC)
    out = _SC_GATHER(src, table)
    return out.reshape(B, S, D)


# speedup vs baseline: 1.4072x; 1.0062x over previous
import functools

import jax
import jax.numpy as jnp
from jax import lax
from jax.experimental import pallas as pl
from jax.experimental.pallas import tpu as pltpu
from jax.experimental.pallas import tpu_sc as plsc

V = 200
D = 98304
NDEST = 1600
DC = 2048
NCH = D // DC
NT = NDEST * NCH
GRP = 16
NG = NT // GRP
NW = 32
GPW = NG // NW
NBUF = 3


def _make_sc_call():
    mesh = plsc.VectorSubcoreMesh(core_axis_name="c", subcore_axis_name="s")

    @functools.partial(
        pl.kernel,
        mesh=mesh,
        out_type=jax.ShapeDtypeStruct((NT, DC), jnp.float32),
        scratch_types=[
            pltpu.VMEM((GPW * GRP,), jnp.int32),
            pltpu.VMEM((NBUF, GRP, DC), jnp.float32),
            pltpu.SemaphoreType.DMA,
            pltpu.SemaphoreType.DMA,
            pltpu.SemaphoreType.DMA,
            pltpu.SemaphoreType.DMA,
            pltpu.SemaphoreType.DMA,
            pltpu.SemaphoreType.DMA,
        ],
    )
    def sc_gather(idx_hbm, table_hbm, out_hbm, idx_v, bufs, g0, g1, g2, s0, s1, s2):
        gsems = (g0, g1, g2)
        ssems = (s0, s1, s2)
        w = lax.axis_index("s") * 2 + lax.axis_index("c")
        base = w * GPW
        pltpu.sync_copy(idx_hbm.at[pl.ds(base * GRP, GPW * GRP)], idx_v)

        def gather_start(k, b):
            g = k - base
            pltpu.make_async_copy(
                table_hbm.at[idx_v.at[pl.ds(g * GRP, GRP)]], bufs.at[b], gsems[b]
            ).start()

        def gather_wait(k, b):
            g = k - base
            pltpu.make_async_copy(
                table_hbm.at[idx_v.at[pl.ds(g * GRP, GRP)]], bufs.at[b], gsems[b]
            ).wait()

        def scatter_start(k, b):
            pltpu.make_async_copy(
                bufs.at[b], out_hbm.at[pl.ds(k * GRP, GRP)], ssems[b]
            ).start()

        def scatter_wait(k, b):
            pltpu.make_async_copy(
                bufs.at[b], out_hbm.at[pl.ds(k * GRP, GRP)], ssems[b]
            ).wait()

        gather_start(base + 0, 0)
        gather_start(base + 1, 1)
        gather_wait(base + 0, 0)
        scatter_start(base + 0, 0)
        gather_start(base + 2, 2)
        gather_wait(base + 1, 1)
        scatter_start(base + 1, 1)
        scatter_wait(base + 0, 0)
        gather_start(base + 3, 0)
        gather_wait(base + 2, 2)
        scatter_start(base + 2, 2)

        def body(r, carry):
            k0 = base + 3 * r
            for j in range(3):
                k = k0 + j
                bn = (j + 1) % 3
                scatter_wait(k - 2, bn)
                gather_start(k + 1, bn)
                gather_wait(k, j)
                scatter_start(k, j)
            return carry

        lax.fori_loop(1, GPW // 3 - 1, body, 0)

        kf = base + GPW - 3
        scatter_wait(kf - 2, 1)
        gather_start(kf + 1, 1)
        gather_wait(kf, 0)
        scatter_start(kf, 0)

        scatter_wait(kf - 1, 2)
        gather_start(kf + 2, 2)
        gather_wait(kf + 1, 1)
        scatter_start(kf + 1, 1)

        scatter_wait(kf, 0)
        gather_wait(kf + 2, 2)
        scatter_start(kf + 2, 2)

        scatter_wait(kf + 1, 1)
        scatter_wait(kf + 2, 2)

    return sc_gather


_SC_GATHER = _make_sc_call()


def kernel(prefix, embedding):
    B, S = prefix.shape
    idx = prefix.reshape(B * S).astype(jnp.int32)
    src = (idx[:, None] * NCH + jnp.arange(NCH, dtype=jnp.int32)[None, :]).reshape(NT)
    table = embedding.reshape(V * NCH, DC)
    out = _SC_GATHER(src, table)
    return out.reshape(B, S, D)


# speedup vs baseline: 1.4083x; 1.0008x over previous
import functools

import jax
import jax.numpy as jnp
from jax import lax
from jax.experimental import pallas as pl
from jax.experimental.pallas import tpu as pltpu
from jax.experimental.pallas import tpu_sc as plsc

V = 200
D = 98304
NDEST = 1600
DC = 6144
NCH = D // DC
NT = NDEST * NCH
GRP = 8
NG = NT // GRP
NW = 32
GPW = NG // NW
NBUF = 2


def _make_sc_call():
    mesh = plsc.VectorSubcoreMesh(core_axis_name="c", subcore_axis_name="s")

    @functools.partial(
        pl.kernel,
        mesh=mesh,
        out_type=jax.ShapeDtypeStruct((NT, DC), jnp.float32),
        scratch_types=[
            pltpu.VMEM((GPW * GRP,), jnp.int32),
            pltpu.VMEM((NBUF, GRP, DC), jnp.float32),
            pltpu.SemaphoreType.DMA,
            pltpu.SemaphoreType.DMA,
            pltpu.SemaphoreType.DMA,
            pltpu.SemaphoreType.DMA,
        ],
    )
    def sc_gather(idx_hbm, table_hbm, out_hbm, idx_v, bufs, g0, g1, s0, s1):
        gsems = (g0, g1)
        ssems = (s0, s1)
        w = lax.axis_index("s") * 2 + lax.axis_index("c")
        base = w * GPW
        pltpu.sync_copy(idx_hbm.at[pl.ds(base * GRP, GPW * GRP)], idx_v)

        def gather_start(k, b):
            g = k - base
            pltpu.make_async_copy(
                table_hbm.at[idx_v.at[pl.ds(g * GRP, GRP)]], bufs.at[b], gsems[b]
            ).start()

        def gather_wait(k, b):
            g = k - base
            pltpu.make_async_copy(
                table_hbm.at[idx_v.at[pl.ds(g * GRP, GRP)]], bufs.at[b], gsems[b]
            ).wait()

        def scatter_start(k, b):
            pltpu.make_async_copy(
                bufs.at[b], out_hbm.at[pl.ds(k * GRP, GRP)], ssems[b]
            ).start()

        def scatter_wait(k, b):
            pltpu.make_async_copy(
                bufs.at[b], out_hbm.at[pl.ds(k * GRP, GRP)], ssems[b]
            ).wait()

        gather_start(base + 0, 0)
        gather_start(base + 1, 1)
        gather_wait(base + 0, 0)
        scatter_start(base + 0, 0)
        scatter_wait(base + 0, 0)
        gather_start(base + 2, 0)
        gather_wait(base + 1, 1)
        scatter_start(base + 1, 1)

        def body(r, carry):
            k0 = base + 2 * r
            for j in range(2):
                k = k0 + j
                bn = (j + 1) % 2
                scatter_wait(k - 1, bn)
                gather_start(k + 1, bn)
                gather_wait(k, j)
                scatter_start(k, j)
            return carry

        lax.fori_loop(1, GPW // 2 - 1, body, 0)

        kf = base + GPW - 2
        scatter_wait(kf - 1, 1)
        gather_start(kf + 1, 1)
        gather_wait(kf, 0)
        scatter_start(kf, 0)

        scatter_wait(kf, 0)
        gather_wait(kf + 1, 1)
        scatter_start(kf + 1, 1)

        scatter_wait(kf + 1, 1)

    return sc_gather


_SC_GATHER = _make_sc_call()


def kernel(prefix, embedding):
    B, S = prefix.shape
    idx = prefix.reshape(B * S).astype(jnp.int32)
    src = (idx[:, None] * NCH + jnp.arange(NCH, dtype=jnp.int32)[None, :]).reshape(NT)
    table = embedding.reshape(V * NCH, DC)
    out = _SC_GATHER(src, table)
    return out.reshape(B, S, D)


# speedup vs baseline: 1.5907x; 1.1295x over previous
import functools

import jax
import jax.numpy as jnp
from jax import lax
from jax.experimental import pallas as pl
from jax.experimental.pallas import tpu as pltpu
from jax.experimental.pallas import tpu_sc as plsc

V = 200
D = 98304
NDEST = 1600
NPAD = NDEST + 16
CW = 8192
CL = CW // 128
NGRP = D // CW
GRP_PER_SC = NGRP // 2
NTILE = 16
DPT = NDEST // NTILE
FULL = DPT // 16
TAIL = DPT - FULL * 16


def _make_sc_call():
    mesh = plsc.VectorSubcoreMesh(core_axis_name="c", subcore_axis_name="s")

    @functools.partial(
        pl.kernel,
        mesh=mesh,
        out_type=jax.ShapeDtypeStruct((NDEST, NGRP, CL, 128), jnp.float32),
        scratch_types=[
            pltpu.VMEM((NPAD,), jnp.int32),
            pltpu.VMEM_SHARED((V, CL, 128), jnp.float32),
            pltpu.SemaphoreType.DMA,
        ],
    )
    def sc_gather(idx_hbm, table_hbm, out_hbm, idx_v, stage, sem):
        c = lax.axis_index("c")
        s = lax.axis_index("s")
        pltpu.sync_copy(idx_hbm, idx_v)

        def mcast16(d0, gc):
            v16 = idx_v[pl.ds(d0, 16)]
            for j in range(16):
                row = v16[j]
                pltpu.make_async_copy(
                    stage.at[row], out_hbm.at[d0 + j, gc], sem
                ).start()

        for g in range(GRP_PER_SC):
            gc = c * GRP_PER_SC + g

            @pl.when(s < 8)
            def _stage():
                r0 = s * (V // 8)
                pltpu.sync_copy(
                    table_hbm.at[pl.ds(r0, V // 8), gc], stage.at[pl.ds(r0, V // 8)]
                )

            plsc.subcore_barrier()

            def body(i, carry):
                mcast16(s * DPT + i * 16, gc)

                @pl.when(i >= 1)
                def _roll():
                    pltpu.make_async_copy(
                        table_hbm.at[pl.ds(0, 16), gc],
                        out_hbm.at[pl.ds(0, 16), gc],
                        sem,
                    ).wait()

                return carry

            lax.fori_loop(0, FULL, body, 0)

            vt = idx_v[pl.ds(s * DPT + FULL * 16, 16)]
            for j in range(TAIL):
                pltpu.make_async_copy(
                    stage.at[vt[j]], out_hbm.at[s * DPT + FULL * 16 + j, gc], sem
                ).start()

            pltpu.make_async_copy(
                table_hbm.at[pl.ds(0, 16 + TAIL), gc],
                out_hbm.at[pl.ds(0, 16 + TAIL), gc],
                sem,
            ).wait()
            plsc.subcore_barrier()

    return sc_gather


_SC_GATHER = _make_sc_call()


def kernel(prefix, embedding):
    B, S = prefix.shape
    idx = prefix.reshape(B * S).astype(jnp.int32)
    idx = jnp.concatenate([idx, jnp.zeros((NPAD - NDEST,), jnp.int32)])
    table = embedding.reshape(V, NGRP, CL, 128)
    out = _SC_GATHER(idx, table)
    return out.reshape(B, S, D)
